# baseline (speedup 1.0000x reference)
import jax
import jax.numpy as jnp
from jax import lax
from jax.experimental import pallas as pl
from jax.experimental.pallas import tpu as pltpu
from jax.experimental.pallas import tpu_sc as plsc

N = 10000
NB = 32
DIN = 128
D = 64
K = 100
C = 40

NW = 32
NODES_PER_W = 320
BC_NODES = 8
BC_GATHERS = BC_NODES * NB // 128
N_BIG = NODES_PER_W // BC_NODES
LAST_W = N // NODES_PER_W
LAST_NODES = N - LAST_W * NODES_PER_W
LAST_BIG = LAST_NODES // BC_NODES

RG = 5
RB = N // RG

_DN = (((1,), (1,)), ((), ()))


def _proj_body(f_ref, wf_ref, bf_ref, wg_ref, o_ref):
    x = jnp.dot(f_ref[0], wf_ref[...], preferred_element_type=jnp.float32)
    x = jnp.maximum(x + bf_ref[...], 0.0)
    o_ref[...] = jnp.dot(x, wg_ref[...],
                         preferred_element_type=jnp.float32).astype(jnp.bfloat16)


def _mm_body(x_ref, w_ref, o_ref):
    x = x_ref[...].astype(jnp.float32)
    o_ref[...] = jnp.dot(x, w_ref[...],
                         preferred_element_type=jnp.float32).astype(jnp.bfloat16)


def _head_body(x_ref, c_ref, wo_ref, bo_ref, o_ref):
    x = x_ref[...].astype(jnp.float32)
    cen = c_ref[...]
    xsq = jnp.sum(x * x, axis=1, keepdims=True)
    csq = jnp.sum(cen * cen, axis=1)[None, :]
    cross = lax.dot_general(x, cen, _DN,
                            preferred_element_type=jnp.float32)
    sq = xsq + csq - 2.0 * cross
    sim = jnp.sqrt(jnp.maximum(sq, 1e-12))
    logit = jnp.dot(sim, wo_ref[...],
                    preferred_element_type=jnp.float32) + bo_ref[...]
    m = jnp.max(logit, axis=1, keepdims=True)
    lse = jnp.log(jnp.sum(jnp.exp(logit - m), axis=1, keepdims=True)) + m
    o_ref[...] = logit - lse


def _sc_agg_body(msg_hbm, idx_hbm, w_hbm, out_hbm, idx_2d, idx_all, w_all, rows,
                 outbuf, table, gsem0, gsem1, osem0, osem1, tsem):
    sid = lax.axis_index("s")
    wid = sid * 2 + lax.axis_index("c")
    node0 = wid * NODES_PER_W
    vtile = N // 16

    pltpu.async_copy(msg_hbm.at[pl.ds(sid * vtile, vtile)],
                     table.at[pl.ds(sid * vtile, vtile)], tsem)

    n_big = jnp.where(wid == LAST_W, LAST_BIG, N_BIG)

    @pl.when(wid < LAST_W)
    def _():
        pltpu.sync_copy(idx_hbm.at[0, pl.ds(node0, NODES_PER_W)], idx_2d)
        pltpu.sync_copy(w_hbm.at[0, pl.ds(node0, NODES_PER_W)], w_all)

    @pl.when(wid == LAST_W)
    def _():
        pltpu.sync_copy(idx_hbm.at[0, pl.ds(node0, LAST_NODES)],
                        idx_2d.at[pl.ds(0, LAST_NODES)])
        pltpu.sync_copy(w_hbm.at[0, pl.ds(node0, LAST_NODES)],
                        w_all.at[pl.ds(0, LAST_NODES)])

    def flat_body(r, carry):
        for g in range(NB // 16):
            idx_all[pl.ds(r * NB + g * 16, 16)] = idx_2d[r, pl.ds(g * 16, 16)]
        return carry

    lax.fori_loop(0, n_big * BC_NODES, flat_body, 0)

    pltpu.make_async_copy(msg_hbm.at[pl.ds(0, vtile)],
                          table.at[pl.ds(0, vtile)], tsem).wait()
    plsc.subcore_barrier()

    gsems = (gsem0, gsem1)
    osems = (osem0, osem1)

    def fire(bc, b):
        for j in range(BC_GATHERS):
            pltpu.async_copy(
                table.at[idx_all.at[pl.ds((bc * BC_GATHERS + j) * 128, 128)]],
                rows.at[b, j], gsems[b])

    def drain(b):
        for j in range(BC_GATHERS):
            pltpu.make_async_copy(table.at[idx_all.at[pl.ds(0, 128)]],
                                  rows.at[b, j], gsems[b]).wait()

    fire(0, 0)

    def pair_body(i, carry):
        for b in range(2):
            bc = i * 2 + b

            @pl.when(bc + 1 < n_big)
            def _():
                fire(bc + 1, 1 - b)

            drain(b)

            @pl.when(bc >= 2)
            def _():
                pltpu.make_async_copy(outbuf.at[b],
                                      out_hbm.at[pl.ds(0, BC_NODES)],
                                      osems[b]).wait()

            for n in range(BC_NODES):
                acc = [jnp.zeros((16,), jnp.float32) for _ in range(4)]
                for g in range(NB // 16):
                    wvec = w_all[bc * BC_NODES + n, pl.ds(g * 16, 16)]
                    for e in range(16):
                        epos = n * NB + g * 16 + e
                        j, r = epos // 128, epos % 128
                        wv = wvec[e]
                        for h in range(2):
                            raw = rows[b, j, r, pl.ds(h * 32, 32)]
                            lo, hi = plsc.unpack(
                                raw, format=plsc.PackFormat.INTERLEAVED)
                            acc[2 * h] = acc[2 * h] + wv * lo
                            acc[2 * h + 1] = acc[2 * h + 1] + wv * hi
                for h in range(2):
                    outbuf[b, n, pl.ds(h * 32, 32)] = plsc.pack(
                        jnp.maximum(acc[2 * h], 0.0),
                        jnp.maximum(acc[2 * h + 1], 0.0),
                        format=plsc.PackFormat.INTERLEAVED)

            pltpu.async_copy(outbuf.at[b],
                             out_hbm.at[pl.ds(node0 + bc * BC_NODES, BC_NODES)],
                             osems[b])
        return carry

    lax.fori_loop(0, n_big // 2, pair_body, 0)
    for b in range(2):
        pltpu.make_async_copy(outbuf.at[b], out_hbm.at[pl.ds(0, BC_NODES)],
                              osems[b]).wait()


def _make_sc_agg():
    return pl.kernel(
        _sc_agg_body,
        out_type=jax.ShapeDtypeStruct((N, D), jnp.bfloat16),
        mesh=plsc.VectorSubcoreMesh(core_axis_name="c", subcore_axis_name="s"),
        compiler_params=pltpu.CompilerParams(use_tc_tiling_on_sc=False,
                                             needs_layout_passes=False),
        scratch_types=[
            pltpu.VMEM((NODES_PER_W, NB), jnp.int32),
            pltpu.VMEM((NODES_PER_W * NB,), jnp.int32),
            pltpu.VMEM((NODES_PER_W, NB), jnp.float32),
            pltpu.VMEM((2, BC_GATHERS, 128, D), jnp.bfloat16),
            pltpu.VMEM((2, BC_NODES, D), jnp.bfloat16),
            pltpu.VMEM_SHARED((N, D), jnp.bfloat16),
            pltpu.SemaphoreType.DMA,
            pltpu.SemaphoreType.DMA,
            pltpu.SemaphoreType.DMA,
            pltpu.SemaphoreType.DMA,
            pltpu.SemaphoreType.DMA,
        ],
    )


def _proj(f3, W_feat, b_feat, W_gnn0):
    return pl.pallas_call(
        _proj_body,
        grid=(RG,),
        in_specs=[
            pl.BlockSpec((1, RB, DIN), lambda i: (0, i, 0)),
            pl.BlockSpec((DIN, D), lambda i: (0, 0)),
            pl.BlockSpec((1, D), lambda i: (0, 0)),
            pl.BlockSpec((D, D), lambda i: (0, 0)),
        ],
        out_specs=pl.BlockSpec((RB, D), lambda i: (i, 0)),
        out_shape=jax.ShapeDtypeStruct((N, D), jnp.bfloat16),
    )(f3, W_feat, b_feat, W_gnn0)


def _mm(x, W):
    return pl.pallas_call(
        _mm_body,
        grid=(RG,),
        in_specs=[
            pl.BlockSpec((RB, D), lambda i: (i, 0)),
            pl.BlockSpec((D, D), lambda i: (0, 0)),
        ],
        out_specs=pl.BlockSpec((RB, D), lambda i: (i, 0)),
        out_shape=jax.ShapeDtypeStruct((N, D), jnp.bfloat16),
    )(x, W)


def _head(x, cen, W_out, b_out):
    return pl.pallas_call(
        _head_body,
        grid=(RG,),
        in_specs=[
            pl.BlockSpec((RB, D), lambda i: (i, 0)),
            pl.BlockSpec((K, D), lambda i: (0, 0)),
            pl.BlockSpec((K, C), lambda i: (0, 0)),
            pl.BlockSpec((1, C), lambda i: (0, 0)),
        ],
        out_specs=pl.BlockSpec((RB, C), lambda i: (i, 0)),
        out_shape=jax.ShapeDtypeStruct((N, C), jnp.float32),
    )(x, cen, W_out, b_out)


def kernel(adj, weight, features, W_feat, b_feat, W_gnn0, W_gnn1, centroids,
           W_out, b_out):
    idx = adj.astype(jnp.int32)
    w = weight.astype(jnp.float32)

    sc_agg = _make_sc_agg()
    msg0 = _proj(features, W_feat, b_feat.reshape(1, D), W_gnn0)
    x1 = sc_agg(msg0, idx, w)
    msg1 = _mm(x1, W_gnn1)
    x2 = sc_agg(msg1, idx, w)
    return _head(x2, centroids, W_out, b_out.reshape(1, C))


# speedup vs baseline: 1.1285x; 1.1285x over previous
import jax
import jax.numpy as jnp
from jax import lax
from jax.experimental import pallas as pl
from jax.experimental.pallas import tpu as pltpu
from jax.experimental.pallas import tpu_sc as plsc

N = 10000
NB = 32
DIN = 128
D = 64
K = 100
C = 40

NW = 32
NODES_PER_W = 320
BC_NODES = 8
BC_GATHERS = BC_NODES * NB // 128
N_BIG = NODES_PER_W // BC_NODES
LAST_W = N // NODES_PER_W
LAST_NODES = N - LAST_W * NODES_PER_W
LAST_BIG = LAST_NODES // BC_NODES

RG = 5
RB = N // RG

_DN = (((1,), (1,)), ((), ()))


def _proj_body(f_ref, wf_ref, bf_ref, wg_ref, o_ref):
    x = jnp.dot(f_ref[0], wf_ref[...], preferred_element_type=jnp.float32)
    x = jnp.maximum(x + bf_ref[...], 0.0)
    o_ref[...] = jnp.dot(x, wg_ref[...], preferred_element_type=jnp.float32)


def _mm_body(x_ref, w_ref, o_ref):
    o_ref[...] = jnp.dot(x_ref[...], w_ref[...],
                         preferred_element_type=jnp.float32)


def _head_body(x_ref, c_ref, wo_ref, bo_ref, o_ref):
    x = x_ref[...]
    cen = c_ref[...]
    xsq = jnp.sum(x * x, axis=1, keepdims=True)
    csq = jnp.sum(cen * cen, axis=1)[None, :]
    cross = lax.dot_general(x, cen, _DN,
                            preferred_element_type=jnp.float32)
    sq = xsq + csq - 2.0 * cross
    sim = jnp.sqrt(jnp.maximum(sq, 1e-12))
    logit = jnp.dot(sim, wo_ref[...],
                    preferred_element_type=jnp.float32) + bo_ref[...]
    m = jnp.max(logit, axis=1, keepdims=True)
    lse = jnp.log(jnp.sum(jnp.exp(logit - m), axis=1, keepdims=True)) + m
    o_ref[...] = logit - lse


def _sc_agg_body(msg_hbm, idx_hbm, w_hbm, out_hbm, idx_2d, idx_all, w_all, rows,
                 outbuf, table, gsem0, gsem1, osem0, osem1, tsem):
    sid = lax.axis_index("s")
    wid = sid * 2 + lax.axis_index("c")
    node0 = wid * NODES_PER_W
    vtile = N // 16

    pltpu.async_copy(msg_hbm.at[pl.ds(sid * vtile, vtile)],
                     table.at[pl.ds(sid * vtile, vtile)], tsem)

    @pl.when(wid < LAST_W)
    def _():
        pltpu.sync_copy(idx_hbm.at[0, pl.ds(node0, NODES_PER_W)], idx_2d)
        pltpu.sync_copy(w_hbm.at[0, pl.ds(node0, NODES_PER_W)], w_all)

    @pl.when(wid == LAST_W)
    def _():
        pltpu.sync_copy(idx_hbm.at[0, pl.ds(node0, LAST_NODES)],
                        idx_2d.at[pl.ds(0, LAST_NODES)])
        pltpu.sync_copy(w_hbm.at[0, pl.ds(node0, LAST_NODES)],
                        w_all.at[pl.ds(0, LAST_NODES)])

        def zero_body(r, carry):
            for g in range(NB // 16):
                idx_2d[r, pl.ds(g * 16, 16)] = jnp.zeros((16,), jnp.int32)
            return carry

        lax.fori_loop(LAST_NODES, NODES_PER_W, zero_body, 0)

    def flat_body(r, carry):
        for g in range(NB // 16):
            idx_all[pl.ds(r * NB + g * 16, 16)] = idx_2d[r, pl.ds(g * 16, 16)]
        return carry

    lax.fori_loop(0, NODES_PER_W, flat_body, 0)

    pltpu.make_async_copy(msg_hbm.at[pl.ds(0, vtile)],
                          table.at[pl.ds(0, vtile)], tsem).wait()
    plsc.subcore_barrier()

    gsems = (gsem0, gsem1)
    osems = (osem0, osem1)

    def fire(bc, b):
        for j in range(BC_GATHERS):
            pltpu.async_copy(
                table.at[idx_all.at[pl.ds((bc * BC_GATHERS + j) * 128, 128)]],
                rows.at[b, j], gsems[b])

    def drain(b):
        for j in range(BC_GATHERS):
            pltpu.make_async_copy(table.at[idx_all.at[pl.ds(0, 128)]],
                                  rows.at[b, j], gsems[b]).wait()

    fire(0, 0)

    def pair_body(i, carry):
        for b in range(2):
            bc = i * 2 + b

            @pl.when(bc + 1 < N_BIG)
            def _():
                fire(bc + 1, 1 - b)

            drain(b)

            @pl.when(bc >= 2)
            def _():
                pltpu.make_async_copy(outbuf.at[b],
                                      out_hbm.at[pl.ds(0, BC_NODES)],
                                      osems[b]).wait()

            for n in range(BC_NODES):
                acc = [jnp.zeros((16,), jnp.float32) for _ in range(4)]
                for g in range(NB // 16):
                    wvec = w_all[bc * BC_NODES + n, pl.ds(g * 16, 16)]
                    for e in range(16):
                        epos = n * NB + g * 16 + e
                        j, r = epos // 128, epos % 128
                        wv = wvec[e]
                        for k in range(4):
                            acc[k] = acc[k] + wv * rows[b, j, r, pl.ds(k * 16, 16)]
                for k in range(4):
                    outbuf[b, n, pl.ds(k * 16, 16)] = jnp.maximum(acc[k], 0.0)

            nbase = jnp.minimum(node0 + bc * BC_NODES, N)
            pltpu.async_copy(outbuf.at[b],
                             out_hbm.at[pl.ds(nbase, BC_NODES)],
                             osems[b])
        return carry

    lax.fori_loop(0, N_BIG // 2, pair_body, 0)
    for b in range(2):
        pltpu.make_async_copy(outbuf.at[b], out_hbm.at[pl.ds(0, BC_NODES)],
                              osems[b]).wait()


def _make_sc_agg():
    return pl.kernel(
        _sc_agg_body,
        out_type=jax.ShapeDtypeStruct((N + BC_NODES, D), jnp.float32),
        mesh=plsc.VectorSubcoreMesh(core_axis_name="c", subcore_axis_name="s"),
        compiler_params=pltpu.CompilerParams(use_tc_tiling_on_sc=False),
        scratch_types=[
            pltpu.VMEM((NODES_PER_W, NB), jnp.int32),
            pltpu.VMEM((NODES_PER_W * NB,), jnp.int32),
            pltpu.VMEM((NODES_PER_W, NB), jnp.float32),
            pltpu.VMEM((2, BC_GATHERS, 128, D), jnp.float32),
            pltpu.VMEM((2, BC_NODES, D), jnp.float32),
            pltpu.VMEM_SHARED((N, D), jnp.float32),
            pltpu.SemaphoreType.DMA,
            pltpu.SemaphoreType.DMA,
            pltpu.SemaphoreType.DMA,
            pltpu.SemaphoreType.DMA,
            pltpu.SemaphoreType.DMA,
        ],
    )


def _proj(f3, W_feat, b_feat, W_gnn0):
    return pl.pallas_call(
        _proj_body,
        grid=(RG,),
        in_specs=[
            pl.BlockSpec((1, RB, DIN), lambda i: (0, i, 0)),
            pl.BlockSpec((DIN, D), lambda i: (0, 0)),
            pl.BlockSpec((1, D), lambda i: (0, 0)),
            pl.BlockSpec((D, D), lambda i: (0, 0)),
        ],
        out_specs=pl.BlockSpec((RB, D), lambda i: (i, 0)),
        out_shape=jax.ShapeDtypeStruct((N, D), jnp.float32),
    )(f3, W_feat, b_feat, W_gnn0)


def _mm(x, W):
    return pl.pallas_call(
        _mm_body,
        grid=(RG,),
        in_specs=[
            pl.BlockSpec((RB, D), lambda i: (i, 0)),
            pl.BlockSpec((D, D), lambda i: (0, 0)),
        ],
        out_specs=pl.BlockSpec((RB, D), lambda i: (i, 0)),
        out_shape=jax.ShapeDtypeStruct((N, D), jnp.float32),
    )(x, W)


def _head(x, cen, W_out, b_out):
    return pl.pallas_call(
        _head_body,
        grid=(RG,),
        in_specs=[
            pl.BlockSpec((RB, D), lambda i: (i, 0)),
            pl.BlockSpec((K, D), lambda i: (0, 0)),
            pl.BlockSpec((K, C), lambda i: (0, 0)),
            pl.BlockSpec((1, C), lambda i: (0, 0)),
        ],
        out_specs=pl.BlockSpec((RB, C), lambda i: (i, 0)),
        out_shape=jax.ShapeDtypeStruct((N, C), jnp.float32),
    )(x, cen, W_out, b_out)


def kernel(adj, weight, features, W_feat, b_feat, W_gnn0, W_gnn1, centroids,
           W_out, b_out):
    idx = adj.astype(jnp.int32)
    w = weight.astype(jnp.float32)

    sc_agg = _make_sc_agg()
    msg0 = _proj(features, W_feat, b_feat.reshape(1, D), W_gnn0)
    x1 = sc_agg(msg0, idx, w)
    msg1 = _mm(x1, W_gnn1)
    x2 = sc_agg(msg1, idx, w)
    return _head(x2, centroids, W_out, b_out.reshape(1, C))
